# baseline (speedup 1.0000x reference)
import jax
import jax.numpy as jnp
from jax import lax
from jax.experimental import pallas as pl
from jax.experimental.pallas import tpu as pltpu
from jax.experimental.pallas import tpu_sc as plsc

_EMB = 32
_G = 512
_NBUF = 5


def _emb_gather_body(idx_hbm, table_hbm, out_hbm, idx_v, rows_v, gsems, wsems):
    nc = 2
    wid = lax.axis_index("s") * nc + lax.axis_index("c")
    nchunk = idx_v.shape[0]
    row_base = wid * (nchunk * _G)

    def gather_desc(j, b):
        return pltpu.make_async_copy(
            table_hbm.at[idx_v.at[j]], rows_v.at[b], gsems.at[b])

    def write_desc(j, b):
        return pltpu.make_async_copy(
            rows_v.at[b], out_hbm.at[pl.ds(row_base + j * _G, _G)],
            wsems.at[b])

    pltpu.sync_copy(idx_hbm.at[wid], idx_v)

    for t in range(_NBUF - 1):
        gather_desc(t, t).start()

    def group(g, carry):
        for b in range(_NBUF):
            j = g * _NBUF + b
            gather_desc(j, b).wait()
            write_desc(j, b).start()
            j2 = j + _NBUF - 1
            b2 = (b + _NBUF - 1) % _NBUF

            @pl.when(j2 < nchunk)
            def _():
                @pl.when(j2 >= _NBUF)
                def _():
                    write_desc(j - 1, b2).wait()

                gather_desc(j2, b2).start()

        return carry

    lax.fori_loop(0, nchunk // _NBUF, group, 0)

    for b in range(_NBUF):
        write_desc(nchunk - _NBUF + b, b).wait()


def kernel(idx, emb_mat):
    b, s = idx.shape
    n = b * s
    info = plsc.get_sparse_core_info()
    nw = info.num_cores * info.num_subcores
    nchunk = n // (nw * _G)
    assert nchunk * nw * _G == n and nchunk % _NBUF == 0
    idx_r = idx.reshape(nw, nchunk, _G).astype(jnp.int32)

    k = pl.kernel(
        _emb_gather_body,
        out_type=jax.ShapeDtypeStruct((n, _EMB), jnp.float32),
        mesh=plsc.VectorSubcoreMesh(core_axis_name="c", subcore_axis_name="s"),
        compiler_params=pltpu.CompilerParams(use_tc_tiling_on_sc=False),
        scratch_types=[
            pltpu.VMEM((nchunk, _G), jnp.int32),
            pltpu.VMEM((_NBUF, _G, _EMB), jnp.float32),
            pltpu.SemaphoreType.DMA((_NBUF,)),
            pltpu.SemaphoreType.DMA((_NBUF,)),
        ],
    )
    out = k(idx_r, emb_mat)
    return out.reshape(b, s, _EMB)


# speedup vs baseline: 1.0005x; 1.0005x over previous
import jax
import jax.numpy as jnp
from jax import lax
from jax.experimental import pallas as pl
from jax.experimental.pallas import tpu as pltpu
from jax.experimental.pallas import tpu_sc as plsc

_EMB = 32
_G = 256
_NBUF = 10
_D = 5


def _emb_gather_body(idx_hbm, table_hbm, out_hbm, idx_v, rows_v, gsems, wsems):
    nc = 2
    wid = lax.axis_index("s") * nc + lax.axis_index("c")
    nchunk = idx_v.shape[0]
    row_base = wid * (nchunk * _G)

    def gather_desc(j, b):
        return pltpu.make_async_copy(
            table_hbm.at[idx_v.at[j]], rows_v.at[b], gsems.at[b])

    def write_desc(j, b):
        return pltpu.make_async_copy(
            rows_v.at[b], out_hbm.at[pl.ds(row_base + j * _G, _G)],
            wsems.at[b])

    pltpu.sync_copy(idx_hbm.at[wid], idx_v)

    for t in range(_D):
        gather_desc(t, t).start()

    def group(g, carry):
        for b in range(_NBUF):
            j = g * _NBUF + b
            gather_desc(j, b).wait()
            write_desc(j, b).start()
            j2 = j + _D
            b2 = (b + _D) % _NBUF

            @pl.when(j2 < nchunk)
            def _():
                @pl.when(j2 >= _NBUF)
                def _():
                    write_desc(j2 - _NBUF, b2).wait()

                gather_desc(j2, b2).start()

        return carry

    lax.fori_loop(0, nchunk // _NBUF, group, 0)

    for b in range(_NBUF):
        write_desc(nchunk - _NBUF + b, b).wait()


def kernel(idx, emb_mat):
    b, s = idx.shape
    n = b * s
    info = plsc.get_sparse_core_info()
    nw = info.num_cores * info.num_subcores
    nchunk = n // (nw * _G)
    assert nchunk * nw * _G == n and nchunk % _NBUF == 0
    idx_r = idx.reshape(nw, nchunk, _G).astype(jnp.int32)

    k = pl.kernel(
        _emb_gather_body,
        out_type=jax.ShapeDtypeStruct((n, _EMB), jnp.float32),
        mesh=plsc.VectorSubcoreMesh(core_axis_name="c", subcore_axis_name="s"),
        compiler_params=pltpu.CompilerParams(use_tc_tiling_on_sc=False),
        scratch_types=[
            pltpu.VMEM((nchunk, _G), jnp.int32),
            pltpu.VMEM((_NBUF, _G, _EMB), jnp.float32),
            pltpu.SemaphoreType.DMA((_NBUF,)),
            pltpu.SemaphoreType.DMA((_NBUF,)),
        ],
    )
    out = k(idx_r, emb_mat)
    return out.reshape(b, s, _EMB)
